# Initial kernel scaffold; baseline (speedup 1.0000x reference)
import jax
import jax.numpy as jnp
from jax.experimental import pallas as pl


def kernel(c, d, tables, W_enc, b_enc, W_fc, b_fc):
    raise NotImplementedError("write your pallas kernel here")


# speedup vs baseline: 6.0659x; 6.0659x over previous
import functools

import jax
import jax.numpy as jnp
from jax import lax
from jax.experimental import pallas as pl
from jax.experimental.pallas import tpu as pltpu
from jax.experimental.pallas import tpu_sc as plsc

F = 26
V = 100000
D = 16
NC_FEAT = 13
CD = 16
OD = 128

L = 16
SC_CORES = 2
SC_SUBCORES = 16
NW = SC_CORES * SC_SUBCORES

CHUNK = 1024
SUB = 128
G = CHUNK // SUB


def _sc_gather(d_flat, table_flat, n_rows):
    rows_per_w = n_rows // NW
    n_chunks = rows_per_w // CHUNK
    assert rows_per_w % CHUNK == 0

    mesh = plsc.VectorSubcoreMesh(core_axis_name="c", subcore_axis_name="s")

    @functools.partial(
        pl.kernel,
        out_type=jax.ShapeDtypeStruct((n_rows, D), jnp.float32),
        mesh=mesh,
        scratch_types=[
            pltpu.VMEM((CHUNK,), jnp.int32),
            pltpu.VMEM((CHUNK, D), jnp.float32),
            pltpu.SemaphoreType.DMA,
        ],
        compiler_params=pltpu.CompilerParams(use_tc_tiling_on_sc=False),
    )
    def k(d_hbm, tbl_hbm, emb_hbm, idx_v, rows_v, sem):
        wid = lax.axis_index("s") * SC_CORES + lax.axis_index("c")
        lane = lax.iota(jnp.int32, L)

        def chunk_body(j, carry):
            base = wid * rows_per_w + j * CHUNK
            pltpu.sync_copy(d_hbm.at[pl.ds(base, CHUNK)], idx_v)

            def adjust(g, carry2):
                r0 = base + g * L
                vals = idx_v[pl.ds(g * L, L)]
                fld = lax.rem(r0 + lane, F)
                idx_v[pl.ds(g * L, L)] = vals + fld * V
                return carry2

            lax.fori_loop(0, CHUNK // L, adjust, 0, unroll=4)

            copies = []
            for g in range(G):
                copies.append(pltpu.async_copy(
                    tbl_hbm.at[idx_v.at[pl.ds(g * SUB, SUB)]],
                    rows_v.at[pl.ds(g * SUB, SUB)],
                    sem,
                ))
            for cp in copies:
                cp.wait()
            pltpu.sync_copy(rows_v, emb_hbm.at[pl.ds(base, CHUNK)])
            return carry

        lax.fori_loop(0, n_chunks, chunk_body, 0)

    return k(d_flat, table_flat)


def _tc_project(emb, c2, w_top, w_enc, b_enc2, w_bot, b_fc2, bm):
    n = emb.shape[0]
    ed = emb.shape[1]

    def body(emb_ref, c_ref, wt_ref, we_ref, be_ref, wb_ref, bf_ref, o_ref):
        enc = jnp.dot(c_ref[...], we_ref[...],
                      preferred_element_type=jnp.float32) + be_ref[...]
        acc = jnp.dot(emb_ref[...], wt_ref[...],
                      preferred_element_type=jnp.float32)
        acc = acc + jnp.dot(enc, wb_ref[...], preferred_element_type=jnp.float32)
        o_ref[...] = acc + bf_ref[...]

    return pl.pallas_call(
        body,
        grid=(n // bm,),
        in_specs=[
            pl.BlockSpec((bm, ed), lambda i: (i, 0)),
            pl.BlockSpec((bm, NC_FEAT), lambda i: (i, 0)),
            pl.BlockSpec((ed, OD), lambda i: (0, 0)),
            pl.BlockSpec((NC_FEAT, CD), lambda i: (0, 0)),
            pl.BlockSpec((1, CD), lambda i: (0, 0)),
            pl.BlockSpec((CD, OD), lambda i: (0, 0)),
            pl.BlockSpec((1, OD), lambda i: (0, 0)),
        ],
        out_specs=pl.BlockSpec((bm, OD), lambda i: (i, 0)),
        out_shape=jax.ShapeDtypeStruct((n, OD), jnp.float32),
    )(emb, c2, w_top, w_enc, b_enc2, w_bot, b_fc2)


def kernel(c, d, tables, W_enc, b_enc, W_fc, b_fc):
    B, S, _ = c.shape
    n_tok = B * S
    n_rows = n_tok * F

    d_flat = d.reshape(n_rows)
    table_flat = tables.reshape(F * V, D)

    emb = _sc_gather(d_flat, table_flat, n_rows)
    emb = emb.reshape(n_tok, F * D)

    c2 = c.reshape(n_tok, NC_FEAT)
    w_top = W_fc[: F * D]
    w_bot = W_fc[F * D:]
    out = _tc_project(emb, c2, w_top, W_enc, b_enc.reshape(1, CD),
                      w_bot, b_fc.reshape(1, OD), bm=1024)
    return out.reshape(B, S, OD)
